# baseline (speedup 1.0000x reference)
import functools

import jax
import jax.numpy as jnp
from jax import lax
from jax.experimental import pallas as pl
from jax.experimental.pallas import tpu as pltpu
from jax.experimental.pallas import tpu_sc as plsc

_NC = 2
_NS = 16
_NW = _NC * _NS
_D = 32
_V = 1000000
_G = _V // 4
_NCOLS = _V // 128
_RD = 6


@functools.lru_cache(maxsize=None)
def _make_relayout():
  mesh = plsc.VectorSubcoreMesh(core_axis_name="c", subcore_axis_name="s")

  @functools.partial(
      pl.kernel,
      mesh=mesh,
      out_type=jax.ShapeDtypeStruct((_V * 32,), jnp.float32),
      scratch_types=[
          pltpu.VMEM((2, 32, 128), jnp.float32),
          [pltpu.VMEM((4096,), jnp.float32) for _ in range(2)],
          pltpu.SemaphoreType.DMA((2,)),
          pltpu.SemaphoreType.DMA((2,)),
          pltpu.VMEM((32, 64), jnp.float32),
          pltpu.VMEM((2048,), jnp.float32),
          pltpu.SemaphoreType.DMA,
      ],
      compiler_params=pltpu.CompilerParams(needs_layout_passes=False),
  )
  def k(tabT_hbm, grp_hbm, ibuf, obufs, isem, osem, tin, tout, tsem):
    wid = lax.axis_index("s") * _NC + lax.axis_index("c")
    q, r = divmod(_NCOLS, _NW)
    lo = wid * q + jnp.minimum(wid, r)
    n = q + jnp.where(wid < r, 1, 0)

    iota = lax.iota(jnp.int32, 16)
    gq = [lax.shift_left(lax.shift_right_logical(iota + 16 * mm, 2), 7)
          + lax.bitwise_and(iota + 16 * mm, 3)
          for mm in range(8)]

    def in_copy(i, bb):
      return pltpu.make_async_copy(
          tabT_hbm.at[:, pl.ds((lo + i) * 128, 128)], ibuf.at[bb],
          isem.at[bb])

    def out_copy(i, bb):
      return pltpu.make_async_copy(
          obufs[bb], grp_hbm.at[pl.ds((lo + i) * 4096, 4096)], osem.at[bb])

    in_copy(0, 0).start()

    def body(i, carry):
      b = lax.rem(i, 2)
      for bb in range(2):
        @pl.when(b == bb)
        def _():
          @pl.when(i + 1 < n)
          def _():
            in_copy(i + 1, 1 - bb).start()
          in_copy(i, bb).wait()
          @pl.when(i >= 2)
          def _():
            out_copy(0, bb).wait()
          def cbody(c, carry2):
            cc = lax.shift_left(c, 2)
            for mm in range(8):
              vals = ibuf[bb, c, pl.ds(16 * mm, 16)]
              plsc.store_scatter(obufs[bb], [gq[mm] + cc], vals)
            return carry2
          lax.fori_loop(0, 32, cbody, 0, unroll=8)
          out_copy(i, bb).start()
      return carry

    lax.fori_loop(0, n, body, 0)
    for bb in range(2):
      out_copy(0, bb).wait()

    @pl.when(wid == 0)
    def _():
      pltpu.sync_copy(tabT_hbm.at[:, pl.ds(_NCOLS * 128, 64)], tin)
      def tbody(c, carry):
        cc = lax.shift_left(c, 2)
        for mm in range(4):
          vals = tin[c, pl.ds(16 * mm, 16)]
          plsc.store_scatter(tout, [gq[mm] + cc], vals)
        return carry
      lax.fori_loop(0, 32, tbody, 0, unroll=8)
      pltpu.async_copy(
          tout, grp_hbm.at[pl.ds(_NCOLS * 4096, 2048)], tsem).wait()

  return k


@functools.lru_cache(maxsize=None)
def _make_gather(R, S):
  i_per_w = R // _NW
  j_per_w = i_per_w * S
  nm = i_per_w // 16
  mesh = plsc.VectorSubcoreMesh(core_axis_name="c", subcore_axis_name="s")

  @functools.partial(
      pl.kernel,
      mesh=mesh,
      out_type=jax.ShapeDtypeStruct((S, _D, R), jnp.float32),
      scratch_types=[
          pltpu.VMEM((j_per_w,), jnp.int32),
          pltpu.VMEM((S, i_per_w), jnp.int32),
          pltpu.VMEM((S, i_per_w), jnp.int32),
          pltpu.VMEM((_RD, i_per_w, 128), jnp.float32),
          pltpu.VMEM((2, _D, i_per_w), jnp.float32),
          pltpu.SemaphoreType.DMA((_RD,)),
          pltpu.SemaphoreType.DMA((2,)),
      ],
      compiler_params=pltpu.CompilerParams(needs_layout_passes=False),
  )
  def k(idx_hbm, grp_hbm, outT_hbm, idxb, idxg, pvec, gbuf, obuf,
        gsem, osem):
    wid = lax.axis_index("s") * _NC + lax.axis_index("c")
    base_i = wid * i_per_w
    pltpu.sync_copy(idx_hbm.at[pl.ds(wid * j_per_w, j_per_w)], idxb)

    iota = lax.iota(jnp.int32, 16)
    jbase = [(iota + 16 * m) * S for m in range(nm)]
    ilvec = [iota + 16 * m for m in range(nm)]

    def prep(jj, carry):
      for m in range(nm):
        rv = plsc.load_gather(idxb, [jbase[m] + jj])
        idxg[jj, pl.ds(16 * m, 16)] = lax.shift_right_logical(rv, 2)
        pvec[jj, pl.ds(16 * m, 16)] = lax.bitwise_and(rv, 3)
      return carry
    lax.fori_loop(0, S, prep, 0, unroll=4)

    def g_copy(jj, bb):
      return pltpu.make_async_copy(
          grp_hbm.at[idxg.at[jj]], gbuf.at[bb], gsem.at[bb])

    def o_copy(jj, ib):
      return pltpu.make_async_copy(
          obuf.at[ib], outT_hbm.at[jj, :, pl.ds(base_i, i_per_w)],
          osem.at[ib])

    for jj in range(_RD - 1):
      g_copy(jj, jj).start()

    def body(jj, carry):
      b = lax.rem(jj, _RD)
      ob = lax.rem(jj, 2)
      nxt = lax.rem(jj + _RD - 1, _RD)
      @pl.when(jj + _RD - 1 < S)
      def _():
        for bb in range(_RD):
          @pl.when(nxt == bb)
          def _():
            g_copy(jj + _RD - 1, bb).start()
      for bb in range(_RD):
        @pl.when(b == bb)
        def _():
          g_copy(jj, bb).wait()
      for ib in range(2):
        @pl.when(ob == ib)
        def _():
          @pl.when(jj >= 2)
          def _():
            o_copy(0, ib).wait()
      bv = jnp.full((16,), b, jnp.int32)
      pv = [pvec[jj, pl.ds(16 * m, 16)] for m in range(nm)]
      def cbody(c, carry2):
        cc = lax.shift_left(c, 2)
        for m in range(nm):
          obuf[ob, c, pl.ds(16 * m, 16)] = plsc.load_gather(
              gbuf, [bv, ilvec[m], pv[m] + cc])
        return carry2
      lax.fori_loop(0, _D, cbody, 0, unroll=8)
      for ib in range(2):
        @pl.when(ob == ib)
        def _():
          o_copy(jj, ib).start()
      return carry

    lax.fori_loop(0, S, body, 0)
    for ib in range(2):
      o_copy(0, ib).wait()

  return k


def kernel(indices, table):
  R, S = indices.shape
  idx1d = indices.reshape(-1).astype(jnp.int32)
  tabT = table.T
  grp1 = _make_relayout()(tabT)
  grp = grp1.reshape(_G, 128)
  outT = _make_gather(R, S)(idx1d, grp)
  return outT.transpose(2, 0, 1)


# speedup vs baseline: 1.0028x; 1.0028x over previous
import functools

import jax
import jax.numpy as jnp
from jax import lax
from jax.experimental import pallas as pl
from jax.experimental.pallas import tpu as pltpu
from jax.experimental.pallas import tpu_sc as plsc

_NC = 2
_NS = 16
_NW = _NC * _NS
_D = 32
_V = 1000000
_G = _V // 4
_NCOLS = _V // 128
_RD = 4


@functools.lru_cache(maxsize=None)
def _make_relayout():
  mesh = plsc.VectorSubcoreMesh(core_axis_name="c", subcore_axis_name="s")

  @functools.partial(
      pl.kernel,
      mesh=mesh,
      out_type=jax.ShapeDtypeStruct((_V * 32,), jnp.float32),
      scratch_types=[
          pltpu.VMEM((2, 32, 128), jnp.float32),
          [pltpu.VMEM((4096,), jnp.float32) for _ in range(2)],
          pltpu.SemaphoreType.DMA((2,)),
          pltpu.SemaphoreType.DMA((2,)),
          pltpu.VMEM((32, 64), jnp.float32),
          pltpu.VMEM((2048,), jnp.float32),
          pltpu.SemaphoreType.DMA,
      ],
      compiler_params=pltpu.CompilerParams(needs_layout_passes=False),
  )
  def k(tabT_hbm, grp_hbm, ibuf, obufs, isem, osem, tin, tout, tsem):
    wid = lax.axis_index("s") * _NC + lax.axis_index("c")
    q, r = divmod(_NCOLS, _NW)
    lo = wid * q + jnp.minimum(wid, r)
    n = q + jnp.where(wid < r, 1, 0)

    iota = lax.iota(jnp.int32, 16)
    gq = [lax.shift_left(lax.shift_right_logical(iota + 16 * mm, 2), 7)
          + lax.bitwise_and(iota + 16 * mm, 3)
          for mm in range(8)]

    def in_copy(i, bb):
      return pltpu.make_async_copy(
          tabT_hbm.at[:, pl.ds((lo + i) * 128, 128)], ibuf.at[bb],
          isem.at[bb])

    def out_copy(i, bb):
      return pltpu.make_async_copy(
          obufs[bb], grp_hbm.at[pl.ds((lo + i) * 4096, 4096)], osem.at[bb])

    in_copy(0, 0).start()

    def body(i, carry):
      b = lax.rem(i, 2)
      for bb in range(2):
        @pl.when(b == bb)
        def _():
          @pl.when(i + 1 < n)
          def _():
            in_copy(i + 1, 1 - bb).start()
          in_copy(i, bb).wait()
          @pl.when(i >= 2)
          def _():
            out_copy(0, bb).wait()
          def cbody(c, carry2):
            cc = lax.shift_left(c, 2)
            for mm in range(8):
              vals = ibuf[bb, c, pl.ds(16 * mm, 16)]
              plsc.store_scatter(obufs[bb], [gq[mm] + cc], vals)
            return carry2
          lax.fori_loop(0, 32, cbody, 0, unroll=16)
          out_copy(i, bb).start()
      return carry

    lax.fori_loop(0, n, body, 0)
    for bb in range(2):
      out_copy(0, bb).wait()

    @pl.when(wid == 0)
    def _():
      pltpu.sync_copy(tabT_hbm.at[:, pl.ds(_NCOLS * 128, 64)], tin)
      def tbody(c, carry):
        cc = lax.shift_left(c, 2)
        for mm in range(4):
          vals = tin[c, pl.ds(16 * mm, 16)]
          plsc.store_scatter(tout, [gq[mm] + cc], vals)
        return carry
      lax.fori_loop(0, 32, tbody, 0, unroll=8)
      pltpu.async_copy(
          tout, grp_hbm.at[pl.ds(_NCOLS * 4096, 2048)], tsem).wait()

  return k


@functools.lru_cache(maxsize=None)
def _make_gather(R, S):
  i_per_w = R // _NW
  j_per_w = i_per_w * S
  nm = i_per_w // 16
  mesh = plsc.VectorSubcoreMesh(core_axis_name="c", subcore_axis_name="s")

  @functools.partial(
      pl.kernel,
      mesh=mesh,
      out_type=jax.ShapeDtypeStruct((S, _D, R), jnp.float32),
      scratch_types=[
          pltpu.VMEM((j_per_w,), jnp.int32),
          pltpu.VMEM((S, i_per_w), jnp.int32),
          pltpu.VMEM((S, i_per_w), jnp.int32),
          pltpu.VMEM((_RD, i_per_w, 128), jnp.float32),
          pltpu.VMEM((2, _D, i_per_w), jnp.float32),
          pltpu.SemaphoreType.DMA((_RD,)),
          pltpu.SemaphoreType.DMA((2,)),
      ],
      compiler_params=pltpu.CompilerParams(needs_layout_passes=False),
  )
  def k(idx_hbm, grp_hbm, outT_hbm, idxb, idxg, pvec, gbuf, obuf,
        gsem, osem):
    wid = lax.axis_index("s") * _NC + lax.axis_index("c")
    base_i = wid * i_per_w
    pltpu.sync_copy(idx_hbm.at[pl.ds(wid * j_per_w, j_per_w)], idxb)

    iota = lax.iota(jnp.int32, 16)
    jbase = [(iota + 16 * m) * S for m in range(nm)]
    ilvec = [iota + 16 * m for m in range(nm)]

    def prep(jj, carry):
      for m in range(nm):
        rv = plsc.load_gather(idxb, [jbase[m] + jj])
        idxg[jj, pl.ds(16 * m, 16)] = lax.shift_right_logical(rv, 2)
        pvec[jj, pl.ds(16 * m, 16)] = lax.bitwise_and(rv, 3)
      return carry
    lax.fori_loop(0, S, prep, 0, unroll=4)

    def g_copy(jj, bb):
      return pltpu.make_async_copy(
          grp_hbm.at[idxg.at[jj]], gbuf.at[bb], gsem.at[bb])

    def o_copy(jj, ib):
      return pltpu.make_async_copy(
          obuf.at[ib], outT_hbm.at[jj, :, pl.ds(base_i, i_per_w)],
          osem.at[ib])

    for jj in range(_RD - 1):
      g_copy(jj, jj).start()

    def body(jj, carry):
      b = lax.rem(jj, _RD)
      ob = lax.rem(jj, 2)
      nxt = lax.rem(jj + _RD - 1, _RD)
      @pl.when(jj + _RD - 1 < S)
      def _():
        for bb in range(_RD):
          @pl.when(nxt == bb)
          def _():
            g_copy(jj + _RD - 1, bb).start()
      for bb in range(_RD):
        @pl.when(b == bb)
        def _():
          g_copy(jj, bb).wait()
      for ib in range(2):
        @pl.when(ob == ib)
        def _():
          @pl.when(jj >= 2)
          def _():
            o_copy(0, ib).wait()
      bv = jnp.full((16,), b, jnp.int32)
      pv = [pvec[jj, pl.ds(16 * m, 16)] for m in range(nm)]
      def cbody(c, carry2):
        cc = lax.shift_left(c, 2)
        for m in range(nm):
          obuf[ob, c, pl.ds(16 * m, 16)] = plsc.load_gather(
              gbuf, [bv, ilvec[m], pv[m] + cc])
        return carry2
      lax.fori_loop(0, _D, cbody, 0, unroll=8)
      for ib in range(2):
        @pl.when(ob == ib)
        def _():
          o_copy(jj, ib).start()
      return carry

    lax.fori_loop(0, S, body, 0)
    for ib in range(2):
      o_copy(0, ib).wait()

  return k


def kernel(indices, table):
  R, S = indices.shape
  idx1d = indices.reshape(-1).astype(jnp.int32)
  tabT = table.T
  grp1 = _make_relayout()(tabT)
  grp = grp1.reshape(_G, 128)
  outT = _make_gather(R, S)(idx1d, grp)
  return outT.transpose(2, 0, 1)


# speedup vs baseline: 1.0050x; 1.0022x over previous
import functools

import jax
import jax.numpy as jnp
from jax import lax
from jax.experimental import pallas as pl
from jax.experimental.pallas import tpu as pltpu
from jax.experimental.pallas import tpu_sc as plsc

_NC = 2
_NS = 16
_NW = _NC * _NS
_D = 32
_V = 1000000
_G = _V // 4
_NCOLS = _V // 128
_RD = 4


@functools.lru_cache(maxsize=None)
def _make_relayout():
  mesh = plsc.VectorSubcoreMesh(core_axis_name="c", subcore_axis_name="s")

  @functools.partial(
      pl.kernel,
      mesh=mesh,
      out_type=jax.ShapeDtypeStruct((_V * 32,), jnp.float32),
      scratch_types=[
          pltpu.VMEM((2, 32, 128), jnp.float32),
          [pltpu.VMEM((4096,), jnp.float32) for _ in range(2)],
          pltpu.SemaphoreType.DMA((2,)),
          pltpu.SemaphoreType.DMA((2,)),
          pltpu.VMEM((32, 64), jnp.float32),
          pltpu.VMEM((2048,), jnp.float32),
          pltpu.SemaphoreType.DMA,
      ],
      compiler_params=pltpu.CompilerParams(needs_layout_passes=False),
  )
  def k(tabT_hbm, grp_hbm, ibuf, obufs, isem, osem, tin, tout, tsem):
    wid = lax.axis_index("s") * _NC + lax.axis_index("c")
    q, r = divmod(_NCOLS, _NW)
    lo = wid * q + jnp.minimum(wid, r)
    n = q + jnp.where(wid < r, 1, 0)

    iota = lax.iota(jnp.int32, 16)
    gq = [lax.shift_left(lax.shift_right_logical(iota + 16 * mm, 2), 7)
          + lax.bitwise_and(iota + 16 * mm, 3)
          for mm in range(8)]

    def in_copy(i, bb):
      return pltpu.make_async_copy(
          tabT_hbm.at[:, pl.ds((lo + i) * 128, 128)], ibuf.at[bb],
          isem.at[bb])

    def out_copy(i, bb):
      return pltpu.make_async_copy(
          obufs[bb], grp_hbm.at[pl.ds((lo + i) * 4096, 4096)], osem.at[bb])

    in_copy(0, 0).start()

    def body(i, carry):
      b = lax.rem(i, 2)
      for bb in range(2):
        @pl.when(b == bb)
        def _():
          @pl.when(i + 1 < n)
          def _():
            in_copy(i + 1, 1 - bb).start()
          in_copy(i, bb).wait()
          @pl.when(i >= 2)
          def _():
            out_copy(0, bb).wait()
          def cbody(c, carry2):
            cc = lax.shift_left(c, 2)
            for mm in range(8):
              vals = ibuf[bb, c, pl.ds(16 * mm, 16)]
              plsc.store_scatter(obufs[bb], [gq[mm] + cc], vals)
            return carry2
          lax.fori_loop(0, 32, cbody, 0, unroll=8)
          out_copy(i, bb).start()
      return carry

    lax.fori_loop(0, n, body, 0)
    for bb in range(2):
      out_copy(0, bb).wait()

    @pl.when(wid == 0)
    def _():
      pltpu.sync_copy(tabT_hbm.at[:, pl.ds(_NCOLS * 128, 64)], tin)
      def tbody(c, carry):
        cc = lax.shift_left(c, 2)
        for mm in range(4):
          vals = tin[c, pl.ds(16 * mm, 16)]
          plsc.store_scatter(tout, [gq[mm] + cc], vals)
        return carry
      lax.fori_loop(0, 32, tbody, 0, unroll=8)
      pltpu.async_copy(
          tout, grp_hbm.at[pl.ds(_NCOLS * 4096, 2048)], tsem).wait()

  return k


@functools.lru_cache(maxsize=None)
def _make_gather(R, S):
  i_per_w = R // _NW
  j_per_w = i_per_w * S
  nm = i_per_w // 16
  mesh = plsc.VectorSubcoreMesh(core_axis_name="c", subcore_axis_name="s")

  @functools.partial(
      pl.kernel,
      mesh=mesh,
      out_type=jax.ShapeDtypeStruct((S, _D, R), jnp.float32),
      scratch_types=[
          pltpu.VMEM((j_per_w,), jnp.int32),
          pltpu.VMEM((S, i_per_w), jnp.int32),
          pltpu.VMEM((S, i_per_w), jnp.int32),
          pltpu.VMEM((_RD, i_per_w, 128), jnp.float32),
          pltpu.VMEM((2, _D, i_per_w), jnp.float32),
          pltpu.SemaphoreType.DMA((_RD,)),
          pltpu.SemaphoreType.DMA((2,)),
      ],
      compiler_params=pltpu.CompilerParams(needs_layout_passes=False),
  )
  def k(idx_hbm, grp_hbm, outT_hbm, idxb, idxg, pvec, gbuf, obuf,
        gsem, osem):
    wid = lax.axis_index("s") * _NC + lax.axis_index("c")
    base_i = wid * i_per_w
    pltpu.sync_copy(idx_hbm.at[pl.ds(wid * j_per_w, j_per_w)], idxb)

    iota = lax.iota(jnp.int32, 16)
    jbase = [(iota + 16 * m) * S for m in range(nm)]
    ilvec = [iota + 16 * m for m in range(nm)]

    def prep(jj, carry):
      for m in range(nm):
        rv = plsc.load_gather(idxb, [jbase[m] + jj])
        idxg[jj, pl.ds(16 * m, 16)] = lax.shift_right_logical(rv, 2)
        pvec[jj, pl.ds(16 * m, 16)] = lax.bitwise_and(rv, 3)
      return carry
    lax.fori_loop(0, S, prep, 0, unroll=4)

    def g_copy(jj, bb):
      return pltpu.make_async_copy(
          grp_hbm.at[idxg.at[jj]], gbuf.at[bb], gsem.at[bb])

    def o_copy(jj, ib):
      return pltpu.make_async_copy(
          obuf.at[ib], outT_hbm.at[jj, :, pl.ds(base_i, i_per_w)],
          osem.at[ib])

    for jj in range(_RD - 1):
      g_copy(jj, jj).start()

    def body(jj, carry):
      b = lax.rem(jj, _RD)
      ob = lax.rem(jj, 2)
      nxt = lax.rem(jj + _RD - 1, _RD)
      @pl.when(jj + _RD - 1 < S)
      def _():
        for bb in range(_RD):
          @pl.when(nxt == bb)
          def _():
            g_copy(jj + _RD - 1, bb).start()
      for bb in range(_RD):
        @pl.when(b == bb)
        def _():
          g_copy(jj, bb).wait()
      for ib in range(2):
        @pl.when(ob == ib)
        def _():
          @pl.when(jj >= 2)
          def _():
            o_copy(0, ib).wait()
      bv = jnp.full((16,), b, jnp.int32)
      pv = [pvec[jj, pl.ds(16 * m, 16)] for m in range(nm)]
      def cbody(c, carry2):
        cc = lax.shift_left(c, 2)
        for m in range(nm):
          obuf[ob, c, pl.ds(16 * m, 16)] = plsc.load_gather(
              gbuf, [bv, ilvec[m], pv[m] + cc])
        return carry2
      lax.fori_loop(0, _D, cbody, 0, unroll=8)
      for ib in range(2):
        @pl.when(ob == ib)
        def _():
          o_copy(jj, ib).start()
      return carry

    lax.fori_loop(0, S, body, 0)
    for ib in range(2):
      o_copy(0, ib).wait()

  return k


def kernel(indices, table):
  R, S = indices.shape
  idx1d = indices.reshape(-1).astype(jnp.int32)
  tabT = table.T
  grp1 = _make_relayout()(tabT)
  grp = grp1.reshape(_G, 128)
  outT = _make_gather(R, S)(idx1d, grp)
  return outT.transpose(2, 0, 1)
